# baseline (speedup 1.0000x reference)
import functools

import jax
import jax.numpy as jnp
from jax import lax
from jax.experimental import pallas as pl
from jax.experimental.pallas import tpu as pltpu
from jax.experimental.pallas import tpu_sc as plsc

F32 = jnp.float32

NC, NS = 2, 16
NW = NC * NS
NN = 10000
NE = 320000
NB = 1000000

EDGE_CHUNK = 128
E_PER_W = 10240
NE_PAD = NW * E_PER_W
N_ECH = E_PER_W // EDGE_CHUNK
N_CHUNKS_TOTAL = NE_PAD // EDGE_CHUNK

SCORE_CHUNK = 1024
B_PER_W = 32768
NB_PAD = NW * B_PER_W
N_SCH = B_PER_W // SCORE_CHUNK

ACC_ROWS = 10240
NEG = -1e30


def _tc(fn, out_shapes, *args):
    n_in = len(args)

    def body(*refs):
        outs = fn(*(r[...] for r in refs[:n_in]))
        if not isinstance(outs, (tuple, list)):
            outs = (outs,)
        for r, o in zip(refs[n_in:], outs):
            r[...] = o

    return pl.pallas_call(
        body,
        out_shape=[jax.ShapeDtypeStruct(s, F32) for s in out_shapes],
    )(*args)


def _prep(ef, Bc, bc, h0, A1):
    BLK = 2048
    LAST = NE // BLK
    P_BLKS = ACC_ROWS // BLK

    def body(e_ref, B_ref, b_ref, h_ref, A_ref, q1_ref, q2_ref, q3_ref, p_ref):
        i = pl.program_id(0)
        rows = i * BLK + lax.broadcasted_iota(jnp.int32, (BLK, 1), 0)
        q = jnp.dot(e_ref[...], B_ref[...], preferred_element_type=F32)
        q = jnp.where(rows < NE, q + b_ref[...], NEG)
        bits = lax.bitcast_convert_type(q, jnp.uint32)
        r16 = (bits + jnp.uint32(0x7FFF) + ((bits >> 16) & 1)) >> 16

        def pack(lo, hi):
            return lax.bitcast_convert_type(lo | (hi << 16), jnp.int32)

        q1_ref[...] = pack(r16[:, 0:32], r16[:, 32:64])
        q2_ref[...] = pack(r16[:, 64:80], r16[:, 80:96])
        q3_ref[...] = pack(r16[:, 96:128], r16[:, 128:160])

        @pl.when(i < P_BLKS)
        def _():
            p_ref[...] = jnp.dot(h_ref[...], A_ref[...],
                                 preferred_element_type=F32)

    qspec = [
        pl.BlockSpec((BLK, 32), lambda i: (i, 0)),
        pl.BlockSpec((BLK, 16), lambda i: (i, 0)),
        pl.BlockSpec((BLK, 32), lambda i: (i, 0)),
    ]
    return pl.pallas_call(
        body,
        grid=(NE_PAD // BLK,),
        in_specs=[
            pl.BlockSpec((BLK, 16), lambda i: (jnp.minimum(i, LAST), 0)),
            pl.BlockSpec((16, 160), lambda i: (0, 0)),
            pl.BlockSpec((1, 160), lambda i: (0, 0)),
            pl.BlockSpec((BLK, 128), lambda i: (jnp.minimum(i, P_BLKS - 1), 0)),
            pl.BlockSpec((128, 64), lambda i: (0, 0)),
        ],
        out_specs=qspec + [
            pl.BlockSpec((BLK, 64), lambda i: (jnp.minimum(i, P_BLKS - 1), 0)),
        ],
        out_shape=[
            jax.ShapeDtypeStruct((NE_PAD, 32), jnp.int32),
            jax.ShapeDtypeStruct((NE_PAD, 16), jnp.int32),
            jax.ShapeDtypeStruct((NE_PAD, 32), jnp.int32),
            jax.ShapeDtypeStruct((ACC_ROWS, 64), F32),
        ],
    )(ef, Bc, bc, h0, A1)


def _unpack_accum(pb, qb, r, g, D):
    xi = qb[r, pl.ds(g * 16, 16)]
    lo = plsc.bitcast(lax.shift_left(xi, 16), F32)
    hi = plsc.bitcast(jnp.bitwise_and(xi, jnp.int32(-65536)), F32)
    sl_lo = pl.ds(g * 16, 16)
    sl_hi = pl.ds(D // 2 + g * 16, 16)
    pb[r, sl_lo] = jnp.maximum(pb[r, sl_lo] + lo, 0.0)
    pb[r, sl_hi] = jnp.maximum(pb[r, sl_hi] + hi, 0.0)


@functools.cache
def _edge_kernel(D):
    mesh = plsc.VectorSubcoreMesh(
        core_axis_name="c", subcore_axis_name="s", num_cores=NC, num_subcores=NS)
    rows_per_tile = ACC_ROWS // NS

    @functools.partial(
        pl.kernel,
        out_type=jax.ShapeDtypeStruct((NC, ACC_ROWS, D), F32),
        mesh=mesh,
        compiler_params=pltpu.CompilerParams(
            use_tc_tiling_on_sc=False, needs_layout_passes=False,
            skip_device_barrier=True),
        scratch_types=[
            pltpu.VMEM_SHARED((ACC_ROWS, D), F32),
            pltpu.VMEM_SHARED((ACC_ROWS, D), F32),
            pltpu.VMEM((2, EDGE_CHUNK), jnp.int32),
            pltpu.VMEM((2, EDGE_CHUNK), jnp.int32),
            pltpu.VMEM((EDGE_CHUNK, D // 2), jnp.int32),
            pltpu.VMEM((EDGE_CHUNK, D // 2), jnp.int32),
            pltpu.VMEM((EDGE_CHUNK, D), F32),
            pltpu.VMEM((EDGE_CHUNK, D), F32),
            pltpu.VMEM((1, EDGE_CHUNK), jnp.int32),
            pltpu.VMEM((1, EDGE_CHUNK), jnp.int32),
            pltpu.SemaphoreType.DMA,
            pltpu.SemaphoreType.DMA,
            pltpu.SemaphoreType.DMA,
            pltpu.SemaphoreType.DMA,
            pltpu.SemaphoreType.DMA,
            pltpu.SemaphoreType.DMA,
            pltpu.SemaphoreType.DMA,
            pltpu.SemaphoreType.DMA,
        ],
    )
    def k(p_hbm, q_hbm, idx_hbm, zeros_hbm, out_hbm,
          acc, ptab, i0, i1, q0, q1, p0, p1, s0, s1,
          ai0, ai1, aq0, aq1, ag0, ag1, as0, as1):
        c = lax.axis_index("c")
        s = lax.axis_index("s")
        wid = s * NC + c
        r0 = s * rows_per_tile
        pltpu.sync_copy(zeros_hbm, acc.at[pl.ds(r0, rows_per_tile)])
        pltpu.sync_copy(p_hbm.at[pl.ds(r0, rows_per_tile)],
                        ptab.at[pl.ds(r0, rows_per_tile)])
        plsc.subcore_barrier()

        bufs = ((i0, q0, p0, s0, ai0, aq0, ag0, as0),
                (i1, q1, p1, s1, ai1, aq1, ag1, as1))
        cb = wid * N_ECH

        def idx_copy(ch, b):
            return pltpu.make_async_copy(idx_hbm.at[cb + ch], bufs[b][0], bufs[b][4])

        def q_copy(ch, b):
            return pltpu.make_async_copy(
                q_hbm.at[pl.ds((cb + ch) * EDGE_CHUNK, EDGE_CHUNK)],
                bufs[b][1], bufs[b][5])

        def gather(b):
            return pltpu.make_async_copy(
                ptab.at[bufs[b][0].at[0]], bufs[b][2], bufs[b][6])

        def scatter_start(b):
            pltpu.async_copy(
                bufs[b][2], acc.at[bufs[b][3].at[0]], bufs[b][7], add=True)

        def scatter_wait(b):
            pltpu.make_async_copy(
                bufs[b][2], acc.at[bufs[b][3].at[0]], bufs[b][7]).wait()

        idx_copy(0, 0).start()
        idx_copy(0, 0).wait()
        q_copy(0, 0).start()
        gather(0).start()
        idx_copy(1, 1).start()

        def body(kk, carry):
            for b in range(2):
                ib, qb, pb, sb = bufs[b][0], bufs[b][1], bufs[b][2], bufs[b][3]
                ch = 2 * kk + b
                q_copy(ch, b).wait()
                gather(b).wait()

                @pl.when(ch < N_ECH - 1)
                def _():
                    idx_copy(ch + 1, 1 - b).wait()

                    @pl.when(ch >= 1)
                    def _():
                        scatter_wait(1 - b)

                    q_copy(ch + 1, 1 - b).start()
                    gather(1 - b).start()

                @plsc.parallel_loop(0, EDGE_CHUNK // 4, unroll=8)
                def _(r4):
                    for dr in range(4):
                        r = r4 * 4 + dr
                        for g in range(D // 32):
                            _unpack_accum(pb, qb, r, g, D)

                for v in range(EDGE_CHUNK // 16):
                    sb[0, pl.ds(v * 16, 16)] = ib[1, pl.ds(v * 16, 16)]
                scatter_start(b)

                @pl.when(ch < N_ECH - 2)
                def _():
                    idx_copy(ch + 2, b).start()
            return carry

        lax.fori_loop(0, N_ECH // 2, body, 0)
        scatter_wait(0)
        scatter_wait(1)
        plsc.subcore_barrier()
        pltpu.sync_copy(acc.at[pl.ds(r0, rows_per_tile)],
                        out_hbm.at[c, pl.ds(r0, rows_per_tile)])

    return k


@functools.cache
def _score_kernel():
    mesh = plsc.VectorSubcoreMesh(
        core_axis_name="c", subcore_axis_name="s", num_cores=NC, num_subcores=NS)

    @functools.partial(
        pl.kernel,
        out_type=jax.ShapeDtypeStruct((NB_PAD,), F32),
        mesh=mesh,
        compiler_params=pltpu.CompilerParams(
            use_tc_tiling_on_sc=False, needs_layout_passes=False,
            skip_device_barrier=True),
        scratch_types=[
            pltpu.VMEM((NN,), F32),
            pltpu.VMEM((NN,), F32),
            pltpu.VMEM((2, SCORE_CHUNK), jnp.int32),
            pltpu.VMEM((2, SCORE_CHUNK), jnp.int32),
            pltpu.VMEM((SCORE_CHUNK,), F32),
            pltpu.VMEM((SCORE_CHUNK,), F32),
            pltpu.SemaphoreType.DMA,
            pltpu.SemaphoreType.DMA,
            pltpu.SemaphoreType.DMA,
            pltpu.SemaphoreType.DMA,
        ],
    )
    def k(sc_hbm, sv_hbm, e_hbm, out_hbm, tcomp, tvm,
          eb0, eb1, ob0, ob1, ae0, ae1, ao0, ao1):
        c = lax.axis_index("c")
        s = lax.axis_index("s")
        wid = s * NC + c
        cb = wid * N_SCH
        pltpu.sync_copy(sc_hbm, tcomp)
        pltpu.sync_copy(sv_hbm, tvm)

        bufs = ((eb0, ob0, ae0, ao0), (eb1, ob1, ae1, ao1))

        def idx_copy(ch, b):
            return pltpu.make_async_copy(
                e_hbm.at[:, cb + ch], bufs[b][0], bufs[b][2])

        def out_copy(ch, b):
            return pltpu.make_async_copy(
                bufs[b][1],
                out_hbm.at[pl.ds((cb + ch) * SCORE_CHUNK, SCORE_CHUNK)],
                bufs[b][3])

        idx_copy(0, 0).start()
        idx_copy(1, 1).start()

        def body(kk, carry):
            for b in range(2):
                eb, ob = bufs[b][0], bufs[b][1]
                ch = 2 * kk + b
                idx_copy(ch, b).wait()

                @pl.when(ch >= 2)
                def _():
                    out_copy(ch - 2, b).wait()

                @plsc.parallel_loop(0, SCORE_CHUNK // 16, unroll=2)
                def _(j):
                    sl = pl.ds(j * 16, 16)
                    x = (plsc.load_gather(tcomp, [eb[0, sl]])
                         + plsc.load_gather(tvm, [eb[1, sl]]))
                    ob[sl] = 1.0 / (1.0 + jnp.exp(-x))

                out_copy(ch, b).start()

                @pl.when(ch < N_SCH - 2)
                def _():
                    idx_copy(ch + 2, b).start()
            return carry

        lax.fori_loop(0, N_SCH // 2, body, 0)
        out_copy(N_SCH - 2, 0).wait()
        out_copy(N_SCH - 1, 1).wait()

    return k


def kernel(edge_index, component_features, component_edges_features, vm_features, edges,
           msg_W1, msg_b1, app_W1, app_b1,
           msg_W2, msg_b2, app_W2, app_b2,
           msg_W3, msg_b3, app_W3, app_b3,
           mlp_W1, mlp_b1, mlp_W2, mlp_b2):
    src = jnp.pad(edge_index[0], (0, NE_PAD - NE))
    dst = jnp.pad(edge_index[1], (0, NE_PAD - NE))
    idx3 = jnp.stack([src.reshape(N_CHUNKS_TOTAL, EDGE_CHUNK),
                      dst.reshape(N_CHUNKS_TOTAL, EDGE_CHUNK)], axis=1)

    def padc(W, D):
        return jnp.pad(W, ((0, 0), (0, D - W.shape[1])))

    def padr(W, D):
        return jnp.pad(W, ((0, D - W.shape[0]), (0, 0)))

    A1, B1 = padc(msg_W1[:128], 64), padc(msg_W1[128:], 64)
    A2, B2 = padc(msg_W2[:50], 32), padc(msg_W2[50:], 32)
    A3, B3 = msg_W3[:25], msg_W3[25:]
    Bc = jnp.concatenate([B1, B2, B3], axis=1)
    bc = jnp.concatenate(
        [jnp.pad(msg_b1, (0, 14)), jnp.pad(msg_b2, (0, 7)), msg_b3]
    ).reshape(1, 160)
    Aa1, Ba1 = app_W1[:128], padr(app_W1[128:], 64)
    Aa2, Ba2 = app_W2[:50], padr(app_W2[50:], 32)
    Aa3, Ba3 = app_W3[:25], app_W3[25:]

    q1, q2, q3, p1 = _prep(component_edges_features, Bc, bc,
                           component_features, A1)
    zeros64 = jnp.zeros((ACC_ROWS // NS, 64), F32)
    zeros32 = jnp.zeros((ACC_ROWS // NS, 32), F32)

    def dot(a, b):
        return jnp.dot(a, b, preferred_element_type=F32)

    def rowpad(x):
        return jnp.concatenate(
            [x, jnp.zeros((ACC_ROWS - NN, x.shape[1]), F32)])

    def mlp(vm, W1, b1, W2, b2):
        hid = jnp.maximum(dot(vm, W1) + b1, 0.0)
        return jnp.sum(dot(hid, W2) + b2, axis=1, keepdims=True)

    (s_vm,) = _tc(mlp, [(NN, 1)],
                  vm_features, mlp_W1, mlp_b1.reshape(1, -1),
                  mlp_W2, mlp_b2.reshape(1, -1))

    parts1 = _edge_kernel(64)(p1, q1, idx3, zeros64)

    def apply_mid(h, parts, Aa, Ba, ba, Anext):
        n = parts[0, :NN] + parts[1, :NN]
        hn = jnp.maximum(dot(h, Aa) + dot(n, Ba) + ba, 0.0)
        return hn, rowpad(dot(hn, Anext))

    h1, p2 = _tc(apply_mid, [(NN, 50), (ACC_ROWS, 32)],
                 component_features, parts1, Aa1, Ba1, app_b1.reshape(1, -1), A2)
    parts2 = _edge_kernel(32)(p2, q2, idx3, zeros32)
    h2, p3 = _tc(apply_mid, [(NN, 25), (ACC_ROWS, 64)],
                 h1, parts2, Aa2, Ba2, app_b2.reshape(1, -1), A3)
    parts3 = _edge_kernel(64)(p3, q3, idx3, zeros64)

    def tail(h, parts, Aa, Ba, ba):
        n = parts[0, :NN] + parts[1, :NN]
        hn = jnp.maximum(dot(h, Aa) + dot(n, Ba) + ba, 0.0)
        return jnp.sum(hn, axis=1, keepdims=True)

    (s_comp,) = _tc(tail, [(NN, 1)],
                    h2, parts3, Aa3, Ba3, app_b3.reshape(1, -1))

    e3 = jnp.pad(edges, ((0, 0), (0, NB_PAD - NB))).reshape(
        2, NB_PAD // SCORE_CHUNK, SCORE_CHUNK)
    logits = _score_kernel()(s_comp.reshape(NN), s_vm.reshape(NN), e3)
    return logits[:NB]


# speedup vs baseline: 1.0120x; 1.0120x over previous
import functools

import jax
import jax.numpy as jnp
from jax import lax
from jax.experimental import pallas as pl
from jax.experimental.pallas import tpu as pltpu
from jax.experimental.pallas import tpu_sc as plsc

F32 = jnp.float32

NC, NS = 2, 16
NW = NC * NS
NN = 10000
NE = 320000
NB = 1000000

EDGE_CHUNK = 128
E_PER_W = 10240
NE_PAD = NW * E_PER_W
N_ECH = E_PER_W // EDGE_CHUNK
N_CHUNKS_TOTAL = NE_PAD // EDGE_CHUNK

SCORE_CHUNK = 1024
B_PER_W = 32768
NB_PAD = NW * B_PER_W
N_SCH = B_PER_W // SCORE_CHUNK

ACC_ROWS = 10240
NEG = -1e30


def _tc(fn, out_shapes, *args):
    n_in = len(args)

    def body(*refs):
        outs = fn(*(r[...] for r in refs[:n_in]))
        if not isinstance(outs, (tuple, list)):
            outs = (outs,)
        for r, o in zip(refs[n_in:], outs):
            r[...] = o

    return pl.pallas_call(
        body,
        out_shape=[jax.ShapeDtypeStruct(s, F32) for s in out_shapes],
    )(*args)


def _prep(ef, Bc, bc, h0, A1):
    BLK = 2048
    LAST = NE // BLK
    P_BLKS = ACC_ROWS // BLK

    def body(e_ref, B_ref, b_ref, h_ref, A_ref, q1_ref, q2_ref, q3_ref, p_ref):
        i = pl.program_id(0)
        rows = i * BLK + lax.broadcasted_iota(jnp.int32, (BLK, 1), 0)
        q = jnp.dot(e_ref[...], B_ref[...], preferred_element_type=F32)
        q = jnp.where(rows < NE, q + b_ref[...], NEG)
        bits = lax.bitcast_convert_type(q, jnp.uint32)
        r16 = (bits + jnp.uint32(0x7FFF) + ((bits >> 16) & 1)) >> 16

        def pack(lo, hi):
            return lax.bitcast_convert_type(lo | (hi << 16), jnp.int32)

        q1_ref[...] = pack(r16[:, 0:32], r16[:, 32:64])
        q2_ref[...] = pack(r16[:, 64:80], r16[:, 80:96])
        q3_ref[...] = pack(r16[:, 96:128], r16[:, 128:160])

        @pl.when(i < P_BLKS)
        def _():
            p_ref[...] = jnp.dot(h_ref[...], A_ref[...],
                                 preferred_element_type=F32)

    qspec = [
        pl.BlockSpec((BLK, 32), lambda i: (i, 0)),
        pl.BlockSpec((BLK, 16), lambda i: (i, 0)),
        pl.BlockSpec((BLK, 32), lambda i: (i, 0)),
    ]
    return pl.pallas_call(
        body,
        grid=(NE_PAD // BLK,),
        in_specs=[
            pl.BlockSpec((BLK, 16), lambda i: (jnp.minimum(i, LAST), 0)),
            pl.BlockSpec((16, 160), lambda i: (0, 0)),
            pl.BlockSpec((1, 160), lambda i: (0, 0)),
            pl.BlockSpec((BLK, 128), lambda i: (jnp.minimum(i, P_BLKS - 1), 0)),
            pl.BlockSpec((128, 64), lambda i: (0, 0)),
        ],
        out_specs=qspec + [
            pl.BlockSpec((BLK, 64), lambda i: (jnp.minimum(i, P_BLKS - 1), 0)),
        ],
        out_shape=[
            jax.ShapeDtypeStruct((NE_PAD, 32), jnp.int32),
            jax.ShapeDtypeStruct((NE_PAD, 16), jnp.int32),
            jax.ShapeDtypeStruct((NE_PAD, 32), jnp.int32),
            jax.ShapeDtypeStruct((ACC_ROWS, 64), F32),
        ],
    )(ef, Bc, bc, h0, A1)


def _unpack_accum(pb, qb, r, g, D):
    xi = qb[r, pl.ds(g * 16, 16)]
    lo = plsc.bitcast(lax.shift_left(xi, 16), F32)
    hi = plsc.bitcast(jnp.bitwise_and(xi, jnp.int32(-65536)), F32)
    sl_lo = pl.ds(g * 16, 16)
    sl_hi = pl.ds(D // 2 + g * 16, 16)
    pb[r, sl_lo] = jnp.maximum(pb[r, sl_lo] + lo, 0.0)
    pb[r, sl_hi] = jnp.maximum(pb[r, sl_hi] + hi, 0.0)


@functools.cache
def _edge_kernel(D):
    mesh = plsc.VectorSubcoreMesh(
        core_axis_name="c", subcore_axis_name="s", num_cores=NC, num_subcores=NS)
    rows_per_tile = ACC_ROWS // NS

    @functools.partial(
        pl.kernel,
        out_type=jax.ShapeDtypeStruct((NC, ACC_ROWS, D), F32),
        mesh=mesh,
        compiler_params=pltpu.CompilerParams(
            use_tc_tiling_on_sc=False, needs_layout_passes=False,
            skip_device_barrier=True),
        scratch_types=[
            pltpu.VMEM_SHARED((ACC_ROWS, D), F32),
            pltpu.VMEM_SHARED((ACC_ROWS, D), F32),
            pltpu.VMEM((2, EDGE_CHUNK), jnp.int32),
            pltpu.VMEM((2, EDGE_CHUNK), jnp.int32),
            pltpu.VMEM((EDGE_CHUNK, D // 2), jnp.int32),
            pltpu.VMEM((EDGE_CHUNK, D // 2), jnp.int32),
            pltpu.VMEM((EDGE_CHUNK, D), F32),
            pltpu.VMEM((EDGE_CHUNK, D), F32),
            pltpu.VMEM((1, EDGE_CHUNK), jnp.int32),
            pltpu.VMEM((1, EDGE_CHUNK), jnp.int32),
            pltpu.SemaphoreType.DMA,
            pltpu.SemaphoreType.DMA,
            pltpu.SemaphoreType.DMA,
            pltpu.SemaphoreType.DMA,
            pltpu.SemaphoreType.DMA,
            pltpu.SemaphoreType.DMA,
            pltpu.SemaphoreType.DMA,
            pltpu.SemaphoreType.DMA,
        ],
    )
    def k(p_hbm, q_hbm, idx_hbm, zeros_hbm, out_hbm,
          acc, ptab, i0, i1, q0, q1, p0, p1, s0, s1,
          ai0, ai1, aq0, aq1, ag0, ag1, as0, as1):
        c = lax.axis_index("c")
        s = lax.axis_index("s")
        wid = s * NC + c
        r0 = s * rows_per_tile
        pltpu.sync_copy(zeros_hbm, acc.at[pl.ds(r0, rows_per_tile)])
        pltpu.sync_copy(p_hbm.at[pl.ds(r0, rows_per_tile)],
                        ptab.at[pl.ds(r0, rows_per_tile)])
        plsc.subcore_barrier()

        bufs = ((i0, q0, p0, s0, ai0, aq0, ag0, as0),
                (i1, q1, p1, s1, ai1, aq1, ag1, as1))
        cb = wid * N_ECH

        def idx_copy(ch, b):
            return pltpu.make_async_copy(idx_hbm.at[cb + ch], bufs[b][0], bufs[b][4])

        def q_copy(ch, b):
            return pltpu.make_async_copy(
                q_hbm.at[pl.ds((cb + ch) * EDGE_CHUNK, EDGE_CHUNK)],
                bufs[b][1], bufs[b][5])

        def gather(b):
            return pltpu.make_async_copy(
                ptab.at[bufs[b][0].at[0]], bufs[b][2], bufs[b][6])

        def scatter_start(b):
            pltpu.async_copy(
                bufs[b][2], acc.at[bufs[b][3].at[0]], bufs[b][7], add=True)

        def scatter_wait(b):
            pltpu.make_async_copy(
                bufs[b][2], acc.at[bufs[b][3].at[0]], bufs[b][7]).wait()

        idx_copy(0, 0).start()
        idx_copy(0, 0).wait()
        q_copy(0, 0).start()
        gather(0).start()
        idx_copy(1, 1).start()

        def body(kk, carry):
            for b in range(2):
                ib, qb, pb, sb = bufs[b][0], bufs[b][1], bufs[b][2], bufs[b][3]
                ch = 2 * kk + b
                q_copy(ch, b).wait()
                gather(b).wait()

                @pl.when(ch < N_ECH - 1)
                def _():
                    idx_copy(ch + 1, 1 - b).wait()

                    @pl.when(ch >= 1)
                    def _():
                        scatter_wait(1 - b)

                    q_copy(ch + 1, 1 - b).start()
                    gather(1 - b).start()

                @plsc.parallel_loop(0, EDGE_CHUNK // 4, unroll=4)
                def _(r4):
                    for dr in range(4):
                        r = r4 * 4 + dr
                        for g in range(D // 32):
                            _unpack_accum(pb, qb, r, g, D)

                for v in range(EDGE_CHUNK // 16):
                    sb[0, pl.ds(v * 16, 16)] = ib[1, pl.ds(v * 16, 16)]
                scatter_start(b)

                @pl.when(ch < N_ECH - 2)
                def _():
                    idx_copy(ch + 2, b).start()
            return carry

        lax.fori_loop(0, N_ECH // 2, body, 0)
        scatter_wait(0)
        scatter_wait(1)
        plsc.subcore_barrier()
        pltpu.sync_copy(acc.at[pl.ds(r0, rows_per_tile)],
                        out_hbm.at[c, pl.ds(r0, rows_per_tile)])

    return k


@functools.cache
def _score_kernel():
    mesh = plsc.VectorSubcoreMesh(
        core_axis_name="c", subcore_axis_name="s", num_cores=NC, num_subcores=NS)

    @functools.partial(
        pl.kernel,
        out_type=jax.ShapeDtypeStruct((NB_PAD,), F32),
        mesh=mesh,
        compiler_params=pltpu.CompilerParams(
            use_tc_tiling_on_sc=False, needs_layout_passes=False,
            skip_device_barrier=True),
        scratch_types=[
            pltpu.VMEM((NN,), F32),
            pltpu.VMEM((NN,), F32),
            pltpu.VMEM((2, SCORE_CHUNK), jnp.int32),
            pltpu.VMEM((2, SCORE_CHUNK), jnp.int32),
            pltpu.VMEM((SCORE_CHUNK,), F32),
            pltpu.VMEM((SCORE_CHUNK,), F32),
            pltpu.SemaphoreType.DMA,
            pltpu.SemaphoreType.DMA,
            pltpu.SemaphoreType.DMA,
            pltpu.SemaphoreType.DMA,
        ],
    )
    def k(sc_hbm, sv_hbm, e_hbm, out_hbm, tcomp, tvm,
          eb0, eb1, ob0, ob1, ae0, ae1, ao0, ao1):
        c = lax.axis_index("c")
        s = lax.axis_index("s")
        wid = s * NC + c
        cb = wid * N_SCH
        pltpu.sync_copy(sc_hbm, tcomp)
        pltpu.sync_copy(sv_hbm, tvm)

        bufs = ((eb0, ob0, ae0, ao0), (eb1, ob1, ae1, ao1))

        def idx_copy(ch, b):
            return pltpu.make_async_copy(
                e_hbm.at[:, cb + ch], bufs[b][0], bufs[b][2])

        def out_copy(ch, b):
            return pltpu.make_async_copy(
                bufs[b][1],
                out_hbm.at[pl.ds((cb + ch) * SCORE_CHUNK, SCORE_CHUNK)],
                bufs[b][3])

        idx_copy(0, 0).start()
        idx_copy(1, 1).start()

        def body(kk, carry):
            for b in range(2):
                eb, ob = bufs[b][0], bufs[b][1]
                ch = 2 * kk + b
                idx_copy(ch, b).wait()

                @pl.when(ch >= 2)
                def _():
                    out_copy(ch - 2, b).wait()

                @plsc.parallel_loop(0, SCORE_CHUNK // 16, unroll=2)
                def _(j):
                    sl = pl.ds(j * 16, 16)
                    x = (plsc.load_gather(tcomp, [eb[0, sl]])
                         + plsc.load_gather(tvm, [eb[1, sl]]))
                    ob[sl] = 1.0 / (1.0 + jnp.exp(-x))

                out_copy(ch, b).start()

                @pl.when(ch < N_SCH - 2)
                def _():
                    idx_copy(ch + 2, b).start()
            return carry

        lax.fori_loop(0, N_SCH // 2, body, 0)
        out_copy(N_SCH - 2, 0).wait()
        out_copy(N_SCH - 1, 1).wait()

    return k


def kernel(edge_index, component_features, component_edges_features, vm_features, edges,
           msg_W1, msg_b1, app_W1, app_b1,
           msg_W2, msg_b2, app_W2, app_b2,
           msg_W3, msg_b3, app_W3, app_b3,
           mlp_W1, mlp_b1, mlp_W2, mlp_b2):
    src = jnp.pad(edge_index[0], (0, NE_PAD - NE))
    dst = jnp.pad(edge_index[1], (0, NE_PAD - NE))
    idx3 = jnp.stack([src.reshape(N_CHUNKS_TOTAL, EDGE_CHUNK),
                      dst.reshape(N_CHUNKS_TOTAL, EDGE_CHUNK)], axis=1)

    def padc(W, D):
        return jnp.pad(W, ((0, 0), (0, D - W.shape[1])))

    def padr(W, D):
        return jnp.pad(W, ((0, D - W.shape[0]), (0, 0)))

    A1, B1 = padc(msg_W1[:128], 64), padc(msg_W1[128:], 64)
    A2, B2 = padc(msg_W2[:50], 32), padc(msg_W2[50:], 32)
    A3, B3 = msg_W3[:25], msg_W3[25:]
    Bc = jnp.concatenate([B1, B2, B3], axis=1)
    bc = jnp.concatenate(
        [jnp.pad(msg_b1, (0, 14)), jnp.pad(msg_b2, (0, 7)), msg_b3]
    ).reshape(1, 160)
    Aa1, Ba1 = app_W1[:128], padr(app_W1[128:], 64)
    Aa2, Ba2 = app_W2[:50], padr(app_W2[50:], 32)
    Aa3, Ba3 = app_W3[:25], app_W3[25:]

    q1, q2, q3, p1 = _prep(component_edges_features, Bc, bc,
                           component_features, A1)
    zeros64 = jnp.zeros((ACC_ROWS // NS, 64), F32)
    zeros32 = jnp.zeros((ACC_ROWS // NS, 32), F32)

    def dot(a, b):
        return jnp.dot(a, b, preferred_element_type=F32)

    def rowpad(x):
        return jnp.concatenate(
            [x, jnp.zeros((ACC_ROWS - NN, x.shape[1]), F32)])

    def mlp(vm, W1, b1, W2, b2):
        hid = jnp.maximum(dot(vm, W1) + b1, 0.0)
        return jnp.sum(dot(hid, W2) + b2, axis=1, keepdims=True)

    (s_vm,) = _tc(mlp, [(NN, 1)],
                  vm_features, mlp_W1, mlp_b1.reshape(1, -1),
                  mlp_W2, mlp_b2.reshape(1, -1))

    parts1 = _edge_kernel(64)(p1, q1, idx3, zeros64)

    def apply_mid(h, parts, Aa, Ba, ba, Anext):
        n = parts[0, :NN] + parts[1, :NN]
        hn = jnp.maximum(dot(h, Aa) + dot(n, Ba) + ba, 0.0)
        return hn, rowpad(dot(hn, Anext))

    h1, p2 = _tc(apply_mid, [(NN, 50), (ACC_ROWS, 32)],
                 component_features, parts1, Aa1, Ba1, app_b1.reshape(1, -1), A2)
    parts2 = _edge_kernel(32)(p2, q2, idx3, zeros32)
    h2, p3 = _tc(apply_mid, [(NN, 25), (ACC_ROWS, 64)],
                 h1, parts2, Aa2, Ba2, app_b2.reshape(1, -1), A3)
    parts3 = _edge_kernel(64)(p3, q3, idx3, zeros64)

    def tail(h, parts, Aa, Ba, ba):
        n = parts[0, :NN] + parts[1, :NN]
        hn = jnp.maximum(dot(h, Aa) + dot(n, Ba) + ba, 0.0)
        return jnp.sum(hn, axis=1, keepdims=True)

    (s_comp,) = _tc(tail, [(NN, 1)],
                    h2, parts3, Aa3, Ba3, app_b3.reshape(1, -1))

    e3 = jnp.pad(edges, ((0, 0), (0, NB_PAD - NB))).reshape(
        2, NB_PAD // SCORE_CHUNK, SCORE_CHUNK)
    logits = _score_kernel()(s_comp.reshape(NN), s_vm.reshape(NN), e3)
    return logits[:NB]
